# Initial kernel scaffold; baseline (speedup 1.0000x reference)
import jax
import jax.numpy as jnp
from jax.experimental import pallas as pl


def kernel(preds, gts, gts_normals, A, mask):
    raise NotImplementedError("write your pallas kernel here")


# speedup vs baseline: 1.1631x; 1.1631x over previous
import functools

import jax
import jax.numpy as jnp
from jax.experimental import pallas as pl
from jax.experimental.pallas import tpu as pltpu


def _loss_kernel(preds_ref, predst_ref, gts_ref, nrm_ref, mask_ref, a_ref,
                 out_ref, vt_ref, *, bi, bj, ng):
    ni = pl.program_id(1)
    nj = pl.program_id(2)

    @pl.when(ni == 0)
    def _phase1():
        gx = gts_ref[0, :, 0:1]
        gy = gts_ref[0, :, 1:2]
        gn = gx * gx + gy * gy
        pjx = predst_ref[0, 0:1, pl.ds(nj * bj, bj)]
        pjy = predst_ref[0, 1:2, pl.ds(nj * bj, bj)]
        scores = gn - 2.0 * (gx * pjx + gy * pjy)
        mcol = mask_ref[0, :, 0:1]
        scores = jnp.where(mcol > 0.5, scores, jnp.inf)
        m = jnp.min(scores, axis=0, keepdims=True)
        gi = jax.lax.broadcasted_iota(jnp.int32, (ng, bj), 0)
        idx = jnp.min(jnp.where(scores == m, gi, ng), axis=0, keepdims=True)
        onehot = gi == idx
        nx = nrm_ref[0, :, 0:1]
        ny = nrm_ref[0, :, 1:2]
        vx = jnp.sum(jnp.where(onehot, nx, 0.0), axis=0, keepdims=True)
        vy = jnp.sum(jnp.where(onehot, ny, 0.0), axis=0, keepdims=True)
        vt = -(vx * pjx + vy * pjy)
        vt_ref[0:1, pl.ds(nj * bj, bj)] = vx
        vt_ref[1:2, pl.ds(nj * bj, bj)] = vy
        vt_ref[2:3, pl.ds(nj * bj, bj)] = vt

    @pl.when((ni == 0) & (nj == 0))
    def _init():
        out_ref[0, :, :] = jnp.zeros((1, 1), jnp.float32)

    pix = preds_ref[0, pl.ds(ni * bi, bi), 0:1]
    piy = preds_ref[0, pl.ds(ni * bi, bi), 1:2]
    vx = vt_ref[0:1, pl.ds(nj * bj, bj)]
    vy = vt_ref[1:2, pl.ds(nj * bj, bj)]
    vt = vt_ref[2:3, pl.ds(nj * bj, bj)]
    s = pix * vx + piy * vy + vt
    w = a_ref[0] * s
    out_ref[0, :, :] = out_ref[0, :, :] + jnp.sum(w * w)


def kernel(preds, gts, gts_normals, A, mask):
    B, Np, D = preds.shape
    Ng = gts.shape[1]
    bi, bj = 256, 256
    predst = jnp.transpose(preds, (0, 2, 1))
    maskf = mask.astype(jnp.float32)[:, :, None]
    grid = (B, Np // bi, Np // bj)
    out = pl.pallas_call(
        functools.partial(_loss_kernel, bi=bi, bj=bj, ng=Ng),
        grid=grid,
        in_specs=[
            pl.BlockSpec((1, Np, D), lambda b, i, j: (b, 0, 0)),
            pl.BlockSpec((1, D, Np), lambda b, i, j: (b, 0, 0)),
            pl.BlockSpec((1, Ng, D), lambda b, i, j: (b, 0, 0)),
            pl.BlockSpec((1, Ng, D), lambda b, i, j: (b, 0, 0)),
            pl.BlockSpec((1, Ng, 1), lambda b, i, j: (b, 0, 0)),
            pl.BlockSpec((1, bi, bj), lambda b, i, j: (b, i, j)),
        ],
        out_specs=pl.BlockSpec((1, 1, 1), lambda b, i, j: (b, 0, 0)),
        out_shape=jax.ShapeDtypeStruct((B, 1, 1), jnp.float32),
        scratch_shapes=[pltpu.VMEM((8, Np), jnp.float32)],
        compiler_params=pltpu.CompilerParams(
            dimension_semantics=("arbitrary", "arbitrary", "arbitrary")),
    )(preds, predst, gts, gts_normals, maskf, A)
    return jnp.sum(out)


# speedup vs baseline: 2.4800x; 2.1323x over previous
import functools

import jax
import jax.numpy as jnp
from jax.experimental import pallas as pl
from jax.experimental.pallas import tpu as pltpu


def _loss_kernel(preds_ref, predst_ref, gts_ref, nrm_ref, mask_ref, a_ref,
                 out_ref, vt_ref, *, bi, bj, ng):
    ni = pl.program_id(1)
    nj = pl.program_id(2)

    @pl.when(ni == 0)
    def _phase1():
        gx = gts_ref[0, :, 0:1]
        gy = gts_ref[0, :, 1:2]
        mcol = mask_ref[0, :, 0:1]
        gn = gx * gx + gy * gy + (1.0 - mcol) * 1e30
        pjx = predst_ref[0, 0:1, pl.ds(nj * bj, bj)]
        pjy = predst_ref[0, 1:2, pl.ds(nj * bj, bj)]
        scores = gn - 2.0 * (gx * pjx + gy * pjy)
        m = jnp.min(scores, axis=0, keepdims=True)
        gi = jax.lax.broadcasted_iota(jnp.int32, (ng, bj), 0)
        idx = jnp.min(jnp.where(scores == m, gi, ng), axis=0, keepdims=True)
        onehot = gi == idx
        nx = nrm_ref[0, :, 0:1]
        ny = nrm_ref[0, :, 1:2]
        vx = jnp.sum(jnp.where(onehot, nx, 0.0), axis=0, keepdims=True)
        vy = jnp.sum(jnp.where(onehot, ny, 0.0), axis=0, keepdims=True)
        vt = -(vx * pjx + vy * pjy)
        vt_ref[0:1, pl.ds(nj * bj, bj)] = vx
        vt_ref[1:2, pl.ds(nj * bj, bj)] = vy
        vt_ref[2:3, pl.ds(nj * bj, bj)] = vt

    @pl.when((ni == 0) & (nj == 0))
    def _init():
        out_ref[0, :, :] = jnp.zeros((1, 1), jnp.float32)

    pix = preds_ref[0, pl.ds(ni * bi, bi), 0:1]
    piy = preds_ref[0, pl.ds(ni * bi, bi), 1:2]
    vx = vt_ref[0:1, pl.ds(nj * bj, bj)]
    vy = vt_ref[1:2, pl.ds(nj * bj, bj)]
    vt = vt_ref[2:3, pl.ds(nj * bj, bj)]
    s = pix * vx + piy * vy + vt
    w = a_ref[0] * s
    out_ref[0, :, :] = out_ref[0, :, :] + jnp.sum(w * w)


def kernel(preds, gts, gts_normals, A, mask):
    B, Np, D = preds.shape
    Ng = gts.shape[1]
    bi, bj = 512, 512
    predst = jnp.transpose(preds, (0, 2, 1))
    maskf = mask.astype(jnp.float32)[:, :, None]
    grid = (B, Np // bi, Np // bj)
    out = pl.pallas_call(
        functools.partial(_loss_kernel, bi=bi, bj=bj, ng=Ng),
        grid=grid,
        in_specs=[
            pl.BlockSpec((1, Np, D), lambda b, i, j: (b, 0, 0)),
            pl.BlockSpec((1, D, Np), lambda b, i, j: (b, 0, 0)),
            pl.BlockSpec((1, Ng, D), lambda b, i, j: (b, 0, 0)),
            pl.BlockSpec((1, Ng, D), lambda b, i, j: (b, 0, 0)),
            pl.BlockSpec((1, Ng, 1), lambda b, i, j: (b, 0, 0)),
            pl.BlockSpec((1, bi, bj), lambda b, i, j: (b, i, j)),
        ],
        out_specs=pl.BlockSpec((1, 1, 1), lambda b, i, j: (b, 0, 0)),
        out_shape=jax.ShapeDtypeStruct((B, 1, 1), jnp.float32),
        scratch_shapes=[pltpu.VMEM((8, Np), jnp.float32)],
        compiler_params=pltpu.CompilerParams(
            dimension_semantics=("parallel", "arbitrary", "arbitrary")),
    )(preds, predst, gts, gts_normals, maskf, A)
    return jnp.sum(out)


# speedup vs baseline: 3.0209x; 1.2181x over previous
import functools

import jax
import jax.numpy as jnp
from jax.experimental import pallas as pl
from jax.experimental.pallas import tpu as pltpu


def _loss_kernel(preds_ref, predst_ref, gts_ref, nrm_ref, mask_ref, a_ref,
                 out_ref, vt_ref, *, bi, bj, ng):
    ni = pl.program_id(1)
    nj = pl.program_id(2)

    @pl.when(ni == 0)
    def _phase1():
        gx = gts_ref[0, :, 0:1]
        gy = gts_ref[0, :, 1:2]
        mcol = mask_ref[0, :, 0:1]
        gn = gx * gx + gy * gy + (1.0 - mcol) * 1e30
        pjx = predst_ref[0, 0:1, pl.ds(nj * bj, bj)]
        pjy = predst_ref[0, 1:2, pl.ds(nj * bj, bj)]
        scores = gn - 2.0 * (gx * pjx + gy * pjy)
        m = jnp.min(scores, axis=0, keepdims=True)
        gi = jax.lax.broadcasted_iota(jnp.int32, (ng, bj), 0)
        idx = jnp.min(jnp.where(scores == m, gi, ng), axis=0, keepdims=True)
        onehot = gi == idx
        nx = nrm_ref[0, :, 0:1]
        ny = nrm_ref[0, :, 1:2]
        vx = jnp.sum(jnp.where(onehot, nx, 0.0), axis=0, keepdims=True)
        vy = jnp.sum(jnp.where(onehot, ny, 0.0), axis=0, keepdims=True)
        vt = -(vx * pjx + vy * pjy)
        vt_ref[0:1, pl.ds(nj * bj, bj)] = vx
        vt_ref[1:2, pl.ds(nj * bj, bj)] = vy
        vt_ref[2:3, pl.ds(nj * bj, bj)] = vt

    @pl.when((ni == 0) & (nj == 0))
    def _init():
        out_ref[0, :, :] = jnp.zeros((1, 1), jnp.float32)

    pix = preds_ref[0, pl.ds(ni * bi, bi), 0:1]
    piy = preds_ref[0, pl.ds(ni * bi, bi), 1:2]
    vx = vt_ref[0:1, pl.ds(nj * bj, bj)]
    vy = vt_ref[1:2, pl.ds(nj * bj, bj)]
    vt = vt_ref[2:3, pl.ds(nj * bj, bj)]
    s = pix * vx + piy * vy + vt
    w = a_ref[0] * s
    out_ref[0, :, :] = out_ref[0, :, :] + jnp.sum(w * w)


def kernel(preds, gts, gts_normals, A, mask):
    B, Np, D = preds.shape
    Ng = gts.shape[1]
    bi, bj = 512, 1024
    predst = jnp.transpose(preds, (0, 2, 1))
    maskf = mask.astype(jnp.float32)[:, :, None]
    grid = (B, Np // bi, Np // bj)
    out = pl.pallas_call(
        functools.partial(_loss_kernel, bi=bi, bj=bj, ng=Ng),
        grid=grid,
        in_specs=[
            pl.BlockSpec((1, Np, D), lambda b, i, j: (b, 0, 0)),
            pl.BlockSpec((1, D, Np), lambda b, i, j: (b, 0, 0)),
            pl.BlockSpec((1, Ng, D), lambda b, i, j: (b, 0, 0)),
            pl.BlockSpec((1, Ng, D), lambda b, i, j: (b, 0, 0)),
            pl.BlockSpec((1, Ng, 1), lambda b, i, j: (b, 0, 0)),
            pl.BlockSpec((1, bi, bj), lambda b, i, j: (b, i, j)),
        ],
        out_specs=pl.BlockSpec((1, 1, 1), lambda b, i, j: (b, 0, 0)),
        out_shape=jax.ShapeDtypeStruct((B, 1, 1), jnp.float32),
        scratch_shapes=[pltpu.VMEM((8, Np), jnp.float32)],
        compiler_params=pltpu.CompilerParams(
            dimension_semantics=("parallel", "arbitrary", "arbitrary")),
    )(preds, predst, gts, gts_normals, maskf, A)
    return jnp.sum(out)


# speedup vs baseline: 3.3640x; 1.1136x over previous
import functools

import jax
import jax.numpy as jnp
from jax.experimental import pallas as pl
from jax.experimental.pallas import tpu as pltpu


def _loss_kernel(preds_ref, predst_ref, gts_ref, nrm_ref, mask_ref, a_ref,
                 out_ref, vt_ref, *, bi, bj, ng):
    ni = pl.program_id(1)
    nj = pl.program_id(2)

    @pl.when(ni == 0)
    def _phase1():
        gx = gts_ref[0, :, 0:1]
        gy = gts_ref[0, :, 1:2]
        mcol = mask_ref[0, :, 0:1]
        gn = gx * gx + gy * gy + (1.0 - mcol) * 1e30
        pjx = predst_ref[0, 0:1, pl.ds(nj * bj, bj)]
        pjy = predst_ref[0, 1:2, pl.ds(nj * bj, bj)]
        scores = gn - 2.0 * (gx * pjx + gy * pjy)
        m = jnp.min(scores, axis=0, keepdims=True)
        gi = jax.lax.broadcasted_iota(jnp.int32, (ng, bj), 0)
        idx = jnp.min(jnp.where(scores == m, gi, ng), axis=0, keepdims=True)
        onehot = gi == idx
        nx = nrm_ref[0, :, 0:1]
        ny = nrm_ref[0, :, 1:2]
        vx = jnp.sum(jnp.where(onehot, nx, 0.0), axis=0, keepdims=True)
        vy = jnp.sum(jnp.where(onehot, ny, 0.0), axis=0, keepdims=True)
        vt = -(vx * pjx + vy * pjy)
        vt_ref[0:1, pl.ds(nj * bj, bj)] = vx
        vt_ref[1:2, pl.ds(nj * bj, bj)] = vy
        vt_ref[2:3, pl.ds(nj * bj, bj)] = vt

    @pl.when((ni == 0) & (nj == 0))
    def _init():
        out_ref[0, :, :] = jnp.zeros((1, 1), jnp.float32)

    pix = preds_ref[0, pl.ds(ni * bi, bi), 0:1]
    piy = preds_ref[0, pl.ds(ni * bi, bi), 1:2]
    vx = vt_ref[0:1, pl.ds(nj * bj, bj)]
    vy = vt_ref[1:2, pl.ds(nj * bj, bj)]
    vt = vt_ref[2:3, pl.ds(nj * bj, bj)]
    s = pix * vx + piy * vy + vt
    w = a_ref[0] * s
    out_ref[0, :, :] = out_ref[0, :, :] + jnp.sum(w * w)


def kernel(preds, gts, gts_normals, A, mask):
    B, Np, D = preds.shape
    Ng = gts.shape[1]
    bi, bj = 1024, 1024
    predst = jnp.transpose(preds, (0, 2, 1))
    maskf = mask.astype(jnp.float32)[:, :, None]
    grid = (B, Np // bi, Np // bj)
    out = pl.pallas_call(
        functools.partial(_loss_kernel, bi=bi, bj=bj, ng=Ng),
        grid=grid,
        in_specs=[
            pl.BlockSpec((1, Np, D), lambda b, i, j: (b, 0, 0)),
            pl.BlockSpec((1, D, Np), lambda b, i, j: (b, 0, 0)),
            pl.BlockSpec((1, Ng, D), lambda b, i, j: (b, 0, 0)),
            pl.BlockSpec((1, Ng, D), lambda b, i, j: (b, 0, 0)),
            pl.BlockSpec((1, Ng, 1), lambda b, i, j: (b, 0, 0)),
            pl.BlockSpec((1, bi, bj), lambda b, i, j: (b, i, j)),
        ],
        out_specs=pl.BlockSpec((1, 1, 1), lambda b, i, j: (b, 0, 0)),
        out_shape=jax.ShapeDtypeStruct((B, 1, 1), jnp.float32),
        scratch_shapes=[pltpu.VMEM((8, Np), jnp.float32)],
        compiler_params=pltpu.CompilerParams(
            dimension_semantics=("parallel", "arbitrary", "arbitrary")),
    )(preds, predst, gts, gts_normals, maskf, A)
    return jnp.sum(out)
